# baseline (device time: 201086 ns/iter reference)
import jax
import jax.numpy as jnp
from jax import lax
from jax.experimental import pallas as pl
from jax.experimental.pallas import tpu as pltpu

N_DEV = 16
STREAM_HOPS = ((8, 7), (7, 8), (8, 7), (7, 8))
ROUND_ORDER = ((0, "r"), (1, "l"), (1, "r"), (0, "l"),
               (2, "r"), (3, "l"), (3, "r"), (2, "l"))


def kernel(x, w_mat):
    m_per, k = x.shape
    _, n_per = w_mat.shape
    m_tot = N_DEV * m_per
    mq = m_per // 4

    def body(x_ref, w_ref, out_ref, cq0, cq1, cq2, cq3, wb_ref,
             gbuf_ref, abuf_ref, *sems):
        my = lax.axis_index("i")
        left = (my - 1) % N_DEV
        right = (my + 1) % N_DEV

        bufs = (cq0, cq1, cq2, cq3)
        a_send_sems, a_recv_sems = sems[16], sems[17]

        barrier_sem = pltpu.get_barrier_semaphore()
        for nbr in (left, right):
            pl.semaphore_signal(
                barrier_sem, inc=1,
                device_id=(nbr,), device_id_type=pl.DeviceIdType.MESH,
            )
        pl.semaphore_wait(barrier_sem, 2)

        for qi in range(4):
            bufs[qi][0] = x_ref[pl.ds(qi * mq, mq), :].astype(jnp.bfloat16)

        streams = []
        for qi in range(4):
            rh, lh = STREAM_HOPS[qi]
            streams.append({
                "buf": bufs[qi], "qi": qi, "rh": rh, "lh": lh,
                "r_ss": sems[4 * qi], "r_rs": sems[4 * qi + 1],
                "l_ss": sems[4 * qi + 2], "l_rs": sems[4 * qi + 3],
                "r": [], "l": [],
            })

        def gemm_q(buf, qi, rel):
            origin = (my + rel) % N_DEV
            blk = jnp.dot(buf[rel], wb_ref[...],
                          preferred_element_type=jnp.float32)
            out_ref[pl.ds(origin * m_per + qi * mq, mq), :] = blk
            return jnp.max(jnp.abs(blk))

        def mrc(buf, src_slot, dst_slot, send_sem, recv_sem, dev):
            return pltpu.make_async_remote_copy(
                src_ref=buf.at[src_slot],
                dst_ref=buf.at[dst_slot],
                send_sem=send_sem,
                recv_sem=recv_sem,
                device_id=(dev,),
                device_id_type=pl.DeviceIdType.MESH,
            )

        for si, d in ROUND_ORDER:
            st = streams[si]
            if d == "r":
                r0 = mrc(st["buf"], 0, 15, st["r_ss"].at[0],
                         st["r_rs"].at[0], right)
                st["r"].append(r0)
            else:
                l0 = mrc(st["buf"], 0, 1, st["l_ss"].at[0],
                         st["l_rs"].at[0], left)
                st["l"].append(l0)
            (st["r"][-1] if d == "r" else st["l"][-1]).start()

        wb_ref[...] = w_ref[...].astype(jnp.bfloat16)
        amax = gemm_q(bufs[0], 0, 0)
        for qi in range(1, 4):
            amax = jnp.maximum(amax, gemm_q(bufs[qi], qi, 0))

        for h in range(8):
            arrivals = []
            for si, d in ROUND_ORDER:
                st = streams[si]
                if d == "r":
                    if h < st["rh"]:
                        st["r"][h].wait_recv()
                        if h + 1 < st["rh"]:
                            nxt = mrc(st["buf"], 15 - h, 14 - h,
                                      st["r_ss"].at[h + 1],
                                      st["r_rs"].at[h + 1], right)
                            nxt.start()
                            st["r"].append(nxt)
                        arrivals.append((st["buf"], st["qi"], 15 - h))
                else:
                    if h < st["lh"]:
                        st["l"][h].wait_recv()
                        if h + 1 < st["lh"]:
                            nxt = mrc(st["buf"], 1 + h, 2 + h,
                                      st["l_ss"].at[h + 1],
                                      st["l_rs"].at[h + 1], left)
                            nxt.start()
                            st["l"].append(nxt)
                        arrivals.append((st["buf"], st["qi"], 1 + h))
            for buf, qi, rel in arrivals:
                amax = jnp.maximum(amax, gemm_q(buf, qi, rel))

        for st in streams:
            for r in st["r"] + st["l"]:
                r.wait_send()

        abuf_ref[...] = jnp.full((8, 128), amax, jnp.float32)
        gbuf_ref[0] = jnp.full((8, 128), amax, jnp.float32)

        amax_rdmas = []
        for dist in range(1, N_DEV):
            r = pltpu.make_async_remote_copy(
                src_ref=abuf_ref,
                dst_ref=gbuf_ref.at[N_DEV - dist],
                send_sem=a_send_sems.at[dist - 1],
                recv_sem=a_recv_sems.at[dist - 1],
                device_id=((my + dist) % N_DEV,),
                device_id_type=pl.DeviceIdType.MESH,
            )
            r.start()
            amax_rdmas.append(r)
        for r in amax_rdmas:
            r.wait()

        gmax = jnp.max(gbuf_ref[...])
        scale = gmax / 448.0
        inv_scale = 448.0 / gmax
        q = (out_ref[...] * inv_scale).astype(jnp.float8_e4m3fn)
        out_ref[...] = q.astype(jnp.float32) * scale

    sem_shapes = []
    for rh, lh in STREAM_HOPS:
        sem_shapes += [
            pltpu.SemaphoreType.DMA((rh,)),
            pltpu.SemaphoreType.DMA((rh,)),
            pltpu.SemaphoreType.DMA((lh,)),
            pltpu.SemaphoreType.DMA((lh,)),
        ]
    sem_shapes += [
        pltpu.SemaphoreType.DMA((N_DEV - 1,)),
        pltpu.SemaphoreType.DMA((N_DEV - 1,)),
    ]

    return pl.pallas_call(
        body,
        out_shape=jax.ShapeDtypeStruct((m_tot, n_per), jnp.float32),
        in_specs=[
            pl.BlockSpec(memory_space=pltpu.VMEM),
            pl.BlockSpec(memory_space=pltpu.VMEM),
        ],
        out_specs=pl.BlockSpec(memory_space=pltpu.VMEM),
        scratch_shapes=[
            pltpu.VMEM((N_DEV, m_per // 4, k), jnp.bfloat16),
            pltpu.VMEM((N_DEV, m_per // 4, k), jnp.bfloat16),
            pltpu.VMEM((N_DEV, m_per // 4, k), jnp.bfloat16),
            pltpu.VMEM((N_DEV, m_per // 4, k), jnp.bfloat16),
            pltpu.VMEM((k, n_per), jnp.bfloat16),
            pltpu.VMEM((N_DEV, 8, 128), jnp.float32),
            pltpu.VMEM((8, 128), jnp.float32),
        ] + sem_shapes,
        compiler_params=pltpu.CompilerParams(
            collective_id=0,
            vmem_limit_bytes=100 * 1024 * 1024,
        ),
    )(x, w_mat)


# device time: 198869 ns/iter; 1.0111x vs baseline; 1.0111x over previous
import jax
import jax.numpy as jnp
from jax import lax
from jax.experimental import pallas as pl
from jax.experimental.pallas import tpu as pltpu

N_DEV = 16
RA, LA = 8, 7
RB, LB = 7, 8


def kernel(x, w_mat):
    m_per, k = x.shape
    _, n_per = w_mat.shape
    m_tot = N_DEV * m_per
    mh = m_per // 2

    def body(x_ref, w_ref, out_ref, commA_ref, commB_ref, wb_ref,
             gbuf_ref, abuf_ref,
             ra_ss, ra_rs, rb_ss, rb_rs, la_ss, la_rs, lb_ss, lb_rs,
             a_send_sems, a_recv_sems):
        my = lax.axis_index("i")
        left = (my - 1) % N_DEV
        right = (my + 1) % N_DEV

        barrier_sem = pltpu.get_barrier_semaphore()
        for nbr in (left, right):
            pl.semaphore_signal(
                barrier_sem, inc=1,
                device_id=(nbr,), device_id_type=pl.DeviceIdType.MESH,
            )
        pl.semaphore_wait(barrier_sem, 2)

        commA_ref[0] = x_ref[pl.ds(0, mh), :].astype(jnp.bfloat16)
        commB_ref[0] = x_ref[pl.ds(mh, mh), :].astype(jnp.bfloat16)

        def gemm_half(buf, s, rel):
            origin = (my + rel) % N_DEV
            blk = jnp.dot(buf[rel], wb_ref[...],
                          preferred_element_type=jnp.float32)
            out_ref[pl.ds(origin * m_per + s * mh, mh), :] = blk
            return jnp.max(jnp.abs(blk))

        def mrc(buf, src_slot, dst_slot, send_sem, recv_sem, dev):
            return pltpu.make_async_remote_copy(
                src_ref=buf.at[src_slot],
                dst_ref=buf.at[dst_slot],
                send_sem=send_sem,
                recv_sem=recv_sem,
                device_id=(dev,),
                device_id_type=pl.DeviceIdType.MESH,
            )

        ra = [mrc(commA_ref, 0, 15, ra_ss.at[0], ra_rs.at[0], right)]
        rb = [mrc(commB_ref, 0, 15, rb_ss.at[0], rb_rs.at[0], right)]
        la = [mrc(commA_ref, 0, 1, la_ss.at[0], la_rs.at[0], left)]
        lb = [mrc(commB_ref, 0, 1, lb_ss.at[0], lb_rs.at[0], left)]
        for r in (ra[0], lb[0], rb[0], la[0]):
            r.start()

        wb_ref[...] = w_ref[...].astype(jnp.bfloat16)
        amax = jnp.maximum(gemm_half(commA_ref, 0, 0),
                           gemm_half(commB_ref, 1, 0))

        for h in range(8):
            ra[h].wait_recv()
            if h + 1 < RA:
                nxt = mrc(commA_ref, 15 - h, 14 - h,
                          ra_ss.at[h + 1], ra_rs.at[h + 1], right)
                nxt.start()
                ra.append(nxt)
            lb[h].wait_recv()
            if h + 1 < LB:
                nxt = mrc(commB_ref, 1 + h, 2 + h,
                          lb_ss.at[h + 1], lb_rs.at[h + 1], left)
                nxt.start()
                lb.append(nxt)
            if h < RB:
                rb[h].wait_recv()
                if h + 1 < RB:
                    nxt = mrc(commB_ref, 15 - h, 14 - h,
                              rb_ss.at[h + 1], rb_rs.at[h + 1], right)
                    nxt.start()
                    rb.append(nxt)
            if h < LA:
                la[h].wait_recv()
                if h + 1 < LA:
                    nxt = mrc(commA_ref, 1 + h, 2 + h,
                              la_ss.at[h + 1], la_rs.at[h + 1], left)
                    nxt.start()
                    la.append(nxt)
            amax = jnp.maximum(amax, gemm_half(commA_ref, 0, 15 - h))
            amax = jnp.maximum(amax, gemm_half(commB_ref, 1, 15 - h))
            if h < 7:
                amax = jnp.maximum(amax, gemm_half(commA_ref, 0, 1 + h))
                amax = jnp.maximum(amax, gemm_half(commB_ref, 1, 1 + h))

        for r in ra + rb + la + lb:
            r.wait_send()

        abuf_ref[...] = jnp.full((8, 128), amax, jnp.float32)
        gbuf_ref[0] = jnp.full((8, 128), amax, jnp.float32)

        amax_rdmas = []
        for dist in range(1, N_DEV):
            r = pltpu.make_async_remote_copy(
                src_ref=abuf_ref,
                dst_ref=gbuf_ref.at[N_DEV - dist],
                send_sem=a_send_sems.at[dist - 1],
                recv_sem=a_recv_sems.at[dist - 1],
                device_id=((my + dist) % N_DEV,),
                device_id_type=pl.DeviceIdType.MESH,
            )
            r.start()
            amax_rdmas.append(r)
        for r in amax_rdmas:
            r.wait()

        gmax = jnp.max(gbuf_ref[...])
        scale = gmax / 448.0
        inv_scale = 448.0 / gmax
        q = (out_ref[...] * inv_scale).astype(jnp.float8_e4m3fn)
        out_ref[...] = q.astype(jnp.float32) * scale

    return pl.pallas_call(
        body,
        out_shape=jax.ShapeDtypeStruct((m_tot, n_per), jnp.float32),
        in_specs=[
            pl.BlockSpec(memory_space=pltpu.VMEM),
            pl.BlockSpec(memory_space=pltpu.VMEM),
        ],
        out_specs=pl.BlockSpec(memory_space=pltpu.VMEM),
        scratch_shapes=[
            pltpu.VMEM((N_DEV, mh, k), jnp.bfloat16),
            pltpu.VMEM((N_DEV, mh, k), jnp.bfloat16),
            pltpu.VMEM((k, n_per), jnp.bfloat16),
            pltpu.VMEM((N_DEV, 8, 128), jnp.float32),
            pltpu.VMEM((8, 128), jnp.float32),
            pltpu.SemaphoreType.DMA((RA,)),
            pltpu.SemaphoreType.DMA((RA,)),
            pltpu.SemaphoreType.DMA((RB,)),
            pltpu.SemaphoreType.DMA((RB,)),
            pltpu.SemaphoreType.DMA((LA,)),
            pltpu.SemaphoreType.DMA((LA,)),
            pltpu.SemaphoreType.DMA((LB,)),
            pltpu.SemaphoreType.DMA((LB,)),
            pltpu.SemaphoreType.DMA((N_DEV - 1,)),
            pltpu.SemaphoreType.DMA((N_DEV - 1,)),
        ],
        compiler_params=pltpu.CompilerParams(
            collective_id=0,
            vmem_limit_bytes=100 * 1024 * 1024,
        ),
    )(x, w_mat)
